# baseline (speedup 1.0000x reference)
import functools

import numpy as np
import jax
import jax.numpy as jnp
from jax import lax
from jax.experimental import pallas as pl
from jax.experimental.pallas import tpu as pltpu
from jax.experimental.pallas import tpu_sc as plsc

N_V = 100000
NP = 100352
N_F = 200000
NTILES = 32
FPT = N_F // NTILES
W3 = FPT * 3
FBUF = 18760
CHUNK = 1024
NFULL = 6
REM = 128
NREM = FPT - NFULL * CHUNK
CPT = NP // 16
RW = 8

_COLS = ((0, (1, 2)), (1, (0, 2)), (2, (1, 0)))


def _sc_scatter(vert_pad, zeros, faces_flat):
    mesh = plsc.VectorSubcoreMesh(core_axis_name="c", subcore_axis_name="s")

    @functools.partial(
        pl.kernel,
        mesh=mesh,
        compiler_params=pltpu.CompilerParams(use_tc_tiling_on_sc=False,
                                             needs_layout_passes=False),
        out_type=jax.ShapeDtypeStruct((2, NP, RW), jnp.float32),
        scratch_types=[
            pltpu.VMEM_SHARED((NP, RW), jnp.float32),
            pltpu.VMEM((FBUF,), jnp.int32),
            pltpu.VMEM((CHUNK,), jnp.int32),
            pltpu.VMEM((CHUNK,), jnp.int32),
            pltpu.VMEM((CHUNK,), jnp.int32),
            pltpu.VMEM((CHUNK, RW), jnp.float32),
            pltpu.VMEM((REM,), jnp.int32),
            pltpu.VMEM((REM,), jnp.int32),
            pltpu.VMEM((REM,), jnp.int32),
            pltpu.VMEM((REM, RW), jnp.float32),
        ],
    )
    def body(vp_hbm, z_hbm, f_hbm, out_hbm,
             acc_sh, fbuf, col0, col1, col2, rows, colr0, colr1, colr2, rowsr):
        cid = lax.axis_index("c")
        sid = lax.axis_index("s")
        wid = sid * 2 + cid
        r0 = sid * CPT
        cols = (col0, col1, col2)
        colsr = (colr0, colr1, colr2)

        pltpu.sync_copy(z_hbm.at[pl.ds(r0, CPT)], acc_sh.at[pl.ds(r0, CPT)])

        word0 = wid * W3
        start = word0 // 8 * 8
        shift = word0 - start

        @pl.when(wid < NTILES - 1)
        def _():
            pltpu.sync_copy(f_hbm.at[pl.ds(start, FBUF)], fbuf)

        @pl.when(wid == NTILES - 1)
        def _():
            pltpu.sync_copy(f_hbm.at[pl.ds(start, 18752)],
                            fbuf.at[pl.ds(0, 18752)])

        plsc.subcore_barrier()

        iota3 = jax.lax.iota(jnp.int32, 16) * 3

        def build16(buf, fword, c, mask_n):
            idx = jnp.minimum(shift + fword + iota3 + c, FBUF - 1)
            vals = plsc.load_gather(fbuf, [idx])
            if mask_n < 16:
                lane = jax.lax.iota(jnp.int32, 16)
                vals = jnp.where(lane < mask_n, vals, N_V)
            return vals

        for c in range(NFULL):
            def build_g(g, carry, c=c):
                fword = (c * CHUNK) * 3 + g * 48
                for col in range(3):
                    cols[col][pl.ds(g * 16, 16)] = build16(
                        fbuf, fword, col, 16)
                return carry

            lax.fori_loop(0, CHUNK // 16, build_g, 0)
            for cs, (cd0, cd1) in _COLS:
                pltpu.sync_copy(vp_hbm.at[cols[cs]], rows)
                pltpu.sync_copy(rows, acc_sh.at[cols[cd0]], add=True)
                pltpu.sync_copy(rows, acc_sh.at[cols[cd1]], add=True)

        for g in range(REM // 16):
            fword = (NFULL * CHUNK) * 3 + g * 48
            n = max(0, min(16, NREM - g * 16))
            for col in range(3):
                colsr[col][pl.ds(g * 16, 16)] = build16(fbuf, fword, col, n)
        for cs, (cd0, cd1) in _COLS:
            pltpu.sync_copy(vp_hbm.at[colsr[cs]], rowsr)
            pltpu.sync_copy(rowsr, acc_sh.at[colsr[cd0]], add=True)
            pltpu.sync_copy(rowsr, acc_sh.at[colsr[cd1]], add=True)

        plsc.subcore_barrier()
        pltpu.sync_copy(acc_sh.at[pl.ds(r0, CPT)],
                        out_hbm.at[cid, pl.ds(r0, CPT)])

    return body(vert_pad, zeros, faces_flat)


def _finalize_body(p, v, tdeg, tsum, o):
    x = p[0] + p[1]
    deg = jnp.maximum(
        jnp.dot(x, tdeg[...], preferred_element_type=jnp.float32,
                precision=lax.Precision.HIGHEST), 1.0)
    lap = x / deg - v[...]
    sq = lap * lap
    o[...] = jnp.sqrt(
        jnp.dot(sq, tsum[...], preferred_element_type=jnp.float32,
                precision=lax.Precision.HIGHEST))


def kernel(vert, faces):
    faces_flat = faces.reshape(-1)

    vert_pad = jnp.concatenate(
        [vert, jnp.ones((N_V, 1), jnp.float32),
         jnp.zeros((N_V, RW - 4), jnp.float32)], axis=1)
    vert_pad = jnp.pad(vert_pad, ((0, NP - N_V), (0, 0)))
    zeros = jnp.zeros((NP, RW), jnp.float32)

    part = _sc_scatter(vert_pad, zeros, faces_flat)

    nr = NP * RW // 128
    lanes = np.arange(128)
    tdeg = jnp.asarray(
        (lanes[:, None] == 8 * (lanes[None, :] // 8) + 3).astype(np.float32))
    tsum = jnp.asarray(
        ((lanes[:, None] // 8 == np.arange(16)[None, :])
         & (lanes[:, None] % 8 < 3)).astype(np.float32))
    curve = pl.pallas_call(
        _finalize_body,
        out_shape=jax.ShapeDtypeStruct((nr, 16), jnp.float32),
    )(part.reshape(2, nr, 128), vert_pad.reshape(nr, 128), tdeg, tsum)
    return curve.reshape(NP)[:N_V]
